# baseline (speedup 1.0000x reference)
import math

import jax
import jax.numpy as jnp
from jax import lax
from jax.experimental import pallas as pl
from jax.experimental.pallas import tpu as pltpu
from jax.experimental.pallas import tpu_sc as plsc

VOCAB = 1000000
D_MODEL = 64
COEFF = math.sqrt(D_MODEL)

NC = 2
NS = 16
NW = NC * NS

CHUNK = 128
NBUF = 4


def _table_rowmajor(t_t):
    V = t_t.shape[1]
    BK = 8192
    grid = pl.cdiv(V, BK)

    def body(t_ref, o_ref):
        o_ref[...] = t_ref[...].T

    return pl.pallas_call(
        body,
        grid=(grid,),
        in_specs=[pl.BlockSpec((D_MODEL, BK), lambda i: (0, i))],
        out_specs=pl.BlockSpec((BK, D_MODEL), lambda i: (i, 0)),
        out_shape=jax.ShapeDtypeStruct((V, D_MODEL), jnp.float32),
    )(t_t)


def _out_transposed_scaled(g2):
    J, I, D = g2.shape

    def body(g_ref, y_ref):
        y_ref[0] = g_ref[0].T * COEFF

    return pl.pallas_call(
        body,
        grid=(J,),
        in_specs=[pl.BlockSpec((1, I, D), lambda j: (j, 0, 0))],
        out_specs=pl.BlockSpec((1, D, I), lambda j: (j, 0, 0)),
        out_shape=jax.ShapeDtypeStruct((J, D, I), jnp.float32),
    )(g2)


def _sc_gather(B):
    assert B % (NW * CHUNK) == 0
    b_per_w = B // NW
    G = b_per_w // CHUNK
    assert G % NBUF == 0 and G >= 2 * NBUF

    mesh = plsc.VectorSubcoreMesh(
        core_axis_name="c", subcore_axis_name="s", num_cores=NC, num_subcores=NS
    )

    def body(table_hbm, idx_hbm, out_hbm, idx_v, rows, sem_g, sem_p):
        wid = lax.axis_index("s") * NC + lax.axis_index("c")
        row0 = wid * b_per_w

        pltpu.sync_copy(idx_hbm.at[wid], idx_v)

        def gather_start(g, b):
            pltpu.make_async_copy(
                table_hbm.at[idx_v.at[g]], rows[b], sem_g[b]
            ).start()

        def gather_wait(g, b):
            pltpu.make_async_copy(
                table_hbm.at[idx_v.at[g]], rows[b], sem_g[b]
            ).wait()

        def put_start(g, b):
            pltpu.make_async_copy(
                rows[b], out_hbm.at[pl.ds(row0 + g * CHUNK, CHUNK)], sem_p[b]
            ).start()

        def put_wait(g, b):
            pltpu.make_async_copy(
                rows[b], out_hbm.at[pl.ds(row0 + g * CHUNK, CHUNK)], sem_p[b]
            ).wait()

        def slot(i, t, first, last):
            b = t % NBUF
            gather_wait(i, b)
            put_start(i, b)
            if not first:
                put_wait(i - 2, (t + 2) % NBUF)
            if not last:
                gather_start(i + 2, (t + 2) % NBUF)

        gather_start(0, 0)
        gather_start(1, 1)

        slot(0, 0, True, False)
        slot(1, 1, True, False)
        slot(2, 2, False, False)
        slot(3, 3, False, False)

        def loop_body(j, carry):
            i0 = j * NBUF
            for t in range(NBUF):
                slot(i0 + t, t, False, False)
            return carry

        lax.fori_loop(1, G // NBUF - 1, loop_body, 0)

        i0 = G - NBUF
        slot(i0 + 0, 0, False, False)
        slot(i0 + 1, 1, False, False)
        slot(i0 + 2, 2, False, True)
        slot(i0 + 3, 3, False, True)

        put_wait(G - 2, (G - 2) % NBUF)
        put_wait(G - 1, (G - 1) % NBUF)

    kern = pl.kernel(
        body,
        out_type=jax.ShapeDtypeStruct((B, D_MODEL), jnp.float32),
        mesh=mesh,
        compiler_params=pltpu.CompilerParams(use_tc_tiling_on_sc=False),
        scratch_types=[
            pltpu.VMEM((G, CHUNK), jnp.int32),
            [pltpu.VMEM((CHUNK, D_MODEL), jnp.float32)] * NBUF,
            [pltpu.SemaphoreType.DMA] * NBUF,
            [pltpu.SemaphoreType.DMA] * NBUF,
        ],
    )
    return kern, b_per_w


def kernel(x, table):
    I, J = x.shape
    B = x.size
    x_t = x.T.astype(jnp.int32)
    t_t = table.T

    t_rm = _table_rowmajor(t_t)

    kern, b_per_w = _sc_gather(B)
    idx = x_t.reshape(NW, b_per_w // CHUNK, CHUNK)
    g2 = kern(t_rm, idx).reshape(J, I, D_MODEL)

    y = _out_transposed_scaled(g2)
    return jnp.transpose(y, (2, 0, 1))


# speedup vs baseline: 1.1595x; 1.1595x over previous
import math

import jax
import jax.numpy as jnp
from jax import lax
from jax.experimental import pallas as pl
from jax.experimental.pallas import tpu as pltpu
from jax.experimental.pallas import tpu_sc as plsc

VOCAB = 1000000
D_MODEL = 64
COEFF = math.sqrt(D_MODEL)

NC = 2
NS = 16
LANES = 16
NW = NC * NS

CHUNK = 128
NBUF = 4


def _sc_gather(B):
    assert B % (NW * CHUNK) == 0
    b_per_w = B // NW
    G = b_per_w // CHUNK
    assert G % NBUF == 0 and G >= 2 * NBUF

    mesh = plsc.VectorSubcoreMesh(
        core_axis_name="c", subcore_axis_name="s", num_cores=NC, num_subcores=NS
    )

    def body(table_hbm, idx_hbm, out_hbm, idx_v, rows_in, rows_out, sem_g, sem_p):
        wid = lax.axis_index("s") * NC + lax.axis_index("c")
        row0 = wid * b_per_w

        pltpu.sync_copy(idx_hbm.at[wid], idx_v)

        def gather_start(g, b):
            pltpu.make_async_copy(
                table_hbm.at[idx_v.at[g]], rows_in[b], sem_g[b]
            ).start()

        def gather_wait(g, b):
            pltpu.make_async_copy(
                table_hbm.at[idx_v.at[g]], rows_in[b], sem_g[b]
            ).wait()

        def put_start(g, b):
            pltpu.make_async_copy(
                rows_out[b], out_hbm.at[pl.ds(row0 + g * CHUNK, CHUNK)], sem_p[b]
            ).start()

        def put_wait(g, b):
            pltpu.make_async_copy(
                rows_out[b], out_hbm.at[pl.ds(row0 + g * CHUNK, CHUNK)], sem_p[b]
            ).wait()

        def scale(b):
            src = rows_in[b]
            dst = rows_out[b]

            @plsc.parallel_loop(0, CHUNK, unroll=8)
            def _(r):
                for c in range(D_MODEL // LANES):
                    sl = pl.ds(c * LANES, LANES)
                    dst[r, sl] = src[r, sl] * COEFF

        def slot(i, t, first, last):
            b = t % NBUF
            gather_wait(i, b)
            if not first:
                put_wait(i - NBUF, b)
            scale(b)
            put_start(i, b)
            if not last:
                gather_start(i + NBUF, b)

        for t in range(NBUF):
            gather_start(t, t)
        for t in range(NBUF):
            slot(t, t, True, False)

        def loop_body(j, carry):
            i0 = j * NBUF
            for t in range(NBUF):
                slot(i0 + t, t, False, False)
            return carry

        lax.fori_loop(1, G // NBUF - 1, loop_body, 0)

        i0 = G - NBUF
        for t in range(NBUF):
            slot(i0 + t, t, False, True)
        for t in range(NBUF):
            put_wait(G - NBUF + t, t)

    kern = pl.kernel(
        body,
        out_type=jax.ShapeDtypeStruct((B, D_MODEL), jnp.float32),
        mesh=mesh,
        compiler_params=pltpu.CompilerParams(use_tc_tiling_on_sc=False),
        scratch_types=[
            pltpu.VMEM((G, CHUNK), jnp.int32),
            [pltpu.VMEM((CHUNK, D_MODEL), jnp.float32)] * NBUF,
            [pltpu.VMEM((CHUNK, D_MODEL), jnp.float32)] * NBUF,
            [pltpu.SemaphoreType.DMA] * NBUF,
            [pltpu.SemaphoreType.DMA] * NBUF,
        ],
    )
    return kern, b_per_w


def kernel(x, table):
    I, J = x.shape
    B = x.size
    x_t = x.T.astype(jnp.int32)
    kern, b_per_w = _sc_gather(B)
    idx = x_t.reshape(NW, b_per_w // CHUNK, CHUNK)
    g2 = kern(table, idx).reshape(J, I, D_MODEL)
    return jnp.transpose(g2, (1, 0, 2))
